# Initial kernel scaffold; baseline (speedup 1.0000x reference)
import jax
import jax.numpy as jnp
from jax.experimental import pallas as pl


def kernel(bias_table, rank_idx, file_idx):
    raise NotImplementedError("write your pallas kernel here")


# speedup vs baseline: 1.3017x; 1.3017x over previous
import functools

import jax
import jax.numpy as jnp
from jax import lax
from jax.experimental import pallas as pl
from jax.experimental.pallas import tpu as pltpu
from jax.experimental.pallas import tpu_sc as plsc

NUM_HEADS = 32
NPOS = 64 * 64
LANES = 16
NVEC = NPOS // LANES
ROW_PAD = 256


def _sc_gather(table_pad, rank_flat, file_flat):
    mesh = plsc.VectorSubcoreMesh(core_axis_name="c", subcore_axis_name="s")

    @functools.partial(
        pl.kernel,
        mesh=mesh,
        out_type=jax.ShapeDtypeStruct((NUM_HEADS, NPOS), jnp.float32),
        scratch_types=[
            pltpu.VMEM((ROW_PAD,), jnp.float32),
            pltpu.VMEM((NPOS,), jnp.int32),
            pltpu.VMEM((NPOS,), jnp.int32),
            pltpu.VMEM((NPOS,), jnp.float32),
        ],
        compiler_params=pltpu.CompilerParams(needs_layout_passes=False),
    )
    def run(table_hbm, rank_hbm, file_hbm, out_hbm, table_v, rank_v, file_v, out_v):
        wid = lax.axis_index("s") * 2 + lax.axis_index("c")
        pltpu.sync_copy(table_hbm.at[wid], table_v)
        pltpu.sync_copy(rank_hbm, rank_v)
        pltpu.sync_copy(file_hbm, file_v)

        def body(n, carry):
            base = pl.multiple_of(n * LANES, LANES)
            rv = rank_v[pl.ds(base, LANES)]
            fv = file_v[pl.ds(base, LANES)]
            c = rv * 16 + fv
            out_v[pl.ds(base, LANES)] = plsc.load_gather(table_v, [c])
            return carry

        lax.fori_loop(0, NVEC, body, 0)
        pltpu.sync_copy(out_v, out_hbm.at[wid])

    return run(table_pad, rank_flat, file_flat)


def kernel(bias_table, rank_idx, file_idx):
    tp = jnp.pad(bias_table, ((0, 0), (0, 0), (0, 1))).reshape(NUM_HEADS, 240)
    tp = jnp.pad(tp, ((0, 0), (0, ROW_PAD - 240)))
    out = _sc_gather(tp, rank_idx.reshape(NPOS), file_idx.reshape(NPOS))
    return out.reshape(NUM_HEADS, 64, 64)


# speedup vs baseline: 1.3406x; 1.0298x over previous
import functools

import jax
import jax.numpy as jnp
from jax import lax
from jax.experimental import pallas as pl
from jax.experimental.pallas import tpu as pltpu
from jax.experimental.pallas import tpu_sc as plsc

NUM_HEADS = 32
NPOS = 64 * 64
LANES = 16
UNROLL = 4
NITER = NPOS // (LANES * UNROLL)


def _sc_gather(table, rank_flat, file_flat):
    mesh = plsc.VectorSubcoreMesh(core_axis_name="c", subcore_axis_name="s")

    @functools.partial(
        pl.kernel,
        mesh=mesh,
        out_type=jax.ShapeDtypeStruct((NUM_HEADS, NPOS), jnp.float32),
        scratch_types=[
            pltpu.VMEM((15, 15), jnp.float32),
            pltpu.VMEM((NPOS,), jnp.int32),
            pltpu.VMEM((NPOS,), jnp.int32),
            pltpu.VMEM((NPOS,), jnp.float32),
            pltpu.SemaphoreType.DMA,
            pltpu.SemaphoreType.DMA,
            pltpu.SemaphoreType.DMA,
        ],
        compiler_params=pltpu.CompilerParams(needs_layout_passes=False),
    )
    def run(table_hbm, rank_hbm, file_hbm, out_hbm,
            table_v, rank_v, file_v, out_v, sem_t, sem_r, sem_f):
        wid = lax.axis_index("s") * 2 + lax.axis_index("c")
        ct = pltpu.async_copy(table_hbm.at[wid], table_v, sem_t)
        cr = pltpu.async_copy(rank_hbm, rank_v, sem_r)
        cf = pltpu.async_copy(file_hbm, file_v, sem_f)
        ct.wait()
        cr.wait()
        cf.wait()

        def body(n, carry):
            base = pl.multiple_of(n * (LANES * UNROLL), LANES * UNROLL)
            for u in range(UNROLL):
                off = base + u * LANES
                rv = rank_v[pl.ds(off, LANES)]
                fv = file_v[pl.ds(off, LANES)]
                out_v[pl.ds(off, LANES)] = plsc.load_gather(table_v, [rv, fv])
            return carry

        lax.fori_loop(0, NITER, body, 0)
        pltpu.sync_copy(out_v, out_hbm.at[wid])

    return run(table, rank_flat, file_flat)


def kernel(bias_table, rank_idx, file_idx):
    out = _sc_gather(bias_table, rank_idx.reshape(NPOS), file_idx.reshape(NPOS))
    return out.reshape(NUM_HEADS, 64, 64)


# speedup vs baseline: 1.3686x; 1.0209x over previous
import functools

import jax
import jax.numpy as jnp
from jax import lax
from jax.experimental import pallas as pl
from jax.experimental.pallas import tpu as pltpu
from jax.experimental.pallas import tpu_sc as plsc

NUM_HEADS = 32
NPOS = 64 * 64
LANES = 16
NGROUPS = NPOS // 64


def _sc_gather(table, rank_p, file_p):
    mesh = plsc.VectorSubcoreMesh(core_axis_name="c", subcore_axis_name="s")

    @functools.partial(
        pl.kernel,
        mesh=mesh,
        out_type=jax.ShapeDtypeStruct((NUM_HEADS, NPOS), jnp.float32),
        scratch_types=[
            pltpu.VMEM((15, 15), jnp.float32),
            pltpu.VMEM((NPOS // 4,), jnp.int32),
            pltpu.VMEM((NPOS // 4,), jnp.int32),
            pltpu.VMEM((NPOS,), jnp.float32),
            pltpu.SemaphoreType.DMA,
            pltpu.SemaphoreType.DMA,
            pltpu.SemaphoreType.DMA,
        ],
        compiler_params=pltpu.CompilerParams(needs_layout_passes=False),
    )
    def run(table_hbm, rank_hbm, file_hbm, out_hbm,
            table_v, rank_v, file_v, out_v, sem_t, sem_r, sem_f):
        wid = lax.axis_index("s") * 2 + lax.axis_index("c")
        ct = pltpu.async_copy(table_hbm.at[wid], table_v, sem_t)
        cr = pltpu.async_copy(rank_hbm, rank_v, sem_r)
        cf = pltpu.async_copy(file_hbm, file_v, sem_f)
        ct.wait()
        cr.wait()
        cf.wait()

        for g in range(NGROUPS):
            rw = rank_v[pl.ds(g * LANES, LANES)]
            fw = file_v[pl.ds(g * LANES, LANES)]
            for j in range(4):
                rb = lax.shift_right_logical(rw, 8 * j) & 0xFF
                fb = lax.shift_right_logical(fw, 8 * j) & 0xFF
                out_v[pl.ds(g * 64 + j * LANES, LANES)] = (
                    plsc.load_gather(table_v, [rb, fb]))

        pltpu.sync_copy(out_v, out_hbm.at[wid])

    return run(table, rank_p, file_p)


def _pack_u8(idx):
    v = idx.reshape(-1, 4, 16).swapaxes(1, 2).astype(jnp.uint8)
    return lax.bitcast_convert_type(v, jnp.int32).reshape(NPOS // 4)


def kernel(bias_table, rank_idx, file_idx):
    out = _sc_gather(bias_table, _pack_u8(rank_idx), _pack_u8(file_idx))
    return out.reshape(NUM_HEADS, 64, 64)
